# baseline (speedup 1.0000x reference)
import functools

import jax
import jax.numpy as jnp
from jax import lax
from jax.experimental import pallas as pl
from jax.experimental.pallas import tpu as pltpu
from jax.experimental.pallas import tpu_sc as plsc

T = 50000
NN = 8
H = 128
L = 56
V = 780
B = 1024

TT = 2000
NTILES = T // TT

LP = 128
NC = 2
NS = 16
NW = NC * NS
CHUNK = 200
NCHUNK = 8
PER_W = CHUNK * NCHUNK
T_PAD = NW * PER_W


def _gather_body(curx_hbm, bidx_hbm, emb_hbm, tvs_hbm, x_hbm, tv_hbm,
                 idxx0, idxx1, idxb0, idxb1, xr0, xr1, tr0, tr1,
                 sgx0, sgx1, sgb0, sgb1, ssx0, ssx1, ssb0, ssb1):
    idxx, idxb = [idxx0, idxx1], [idxb0, idxb1]
    xr, tr = [xr0, xr1], [tr0, tr1]
    sgx, sgb = [sgx0, sgx1], [sgb0, sgb1]
    ssx, ssb = [ssx0, ssx1], [ssb0, ssb1]
    wid = lax.axis_index("s") * NC + lax.axis_index("c")
    base0 = wid * PER_W
    gpend = [None, None]
    spend = [None, None]
    for c in range(NCHUNK):
        b = c & 1
        if spend[b] is not None:
            spend[b][0].wait()
            spend[b][1].wait()
            spend[b] = None
        base = base0 + c * CHUNK
        pltpu.sync_copy(curx_hbm.at[pl.ds(base, CHUNK)], idxx[b])
        pltpu.sync_copy(bidx_hbm.at[pl.ds(base, CHUNK)], idxb[b])
        gpend[b] = (pltpu.async_copy(emb_hbm.at[idxx[b]], xr[b], sgx[b]),
                    pltpu.async_copy(tvs_hbm.at[idxb[b]], tr[b], sgb[b]),
                    base)
        pb = 1 - b
        if gpend[pb] is not None:
            gx, gb, pbase = gpend[pb]
            gx.wait()
            gb.wait()
            gpend[pb] = None
            spend[pb] = (
                pltpu.async_copy(xr[pb], x_hbm.at[pl.ds(pbase, CHUNK)], ssx[pb]),
                pltpu.async_copy(tr[pb], tv_hbm.at[pl.ds(pbase, CHUNK)], ssb[pb]))
    b = (NCHUNK - 1) & 1
    gx, gb, pbase = gpend[b]
    gx.wait()
    gb.wait()
    pltpu.sync_copy(xr[b], x_hbm.at[pl.ds(pbase, CHUNK)])
    pltpu.sync_copy(tr[b], tv_hbm.at[pl.ds(pbase, CHUNK)])
    if spend[1 - b] is not None:
        spend[1 - b][0].wait()
        spend[1 - b][1].wait()


def _sc_gather(cur_x_pad, batch_idx_pad, embedding, tree_vecs_pad):
    mesh = plsc.VectorSubcoreMesh(core_axis_name="c", subcore_axis_name="s")
    return pl.kernel(
        _gather_body,
        out_type=[
            jax.ShapeDtypeStruct((T_PAD, H), jnp.float32),
            jax.ShapeDtypeStruct((T_PAD, LP), jnp.float32),
        ],
        mesh=mesh,
        scratch_types=[
            pltpu.VMEM((CHUNK,), jnp.int32),
            pltpu.VMEM((CHUNK,), jnp.int32),
            pltpu.VMEM((CHUNK,), jnp.int32),
            pltpu.VMEM((CHUNK,), jnp.int32),
            pltpu.VMEM((CHUNK, H), jnp.float32),
            pltpu.VMEM((CHUNK, H), jnp.float32),
            pltpu.VMEM((CHUNK, LP), jnp.float32),
            pltpu.VMEM((CHUNK, LP), jnp.float32),
            pltpu.SemaphoreType.DMA,
            pltpu.SemaphoreType.DMA,
            pltpu.SemaphoreType.DMA,
            pltpu.SemaphoreType.DMA,
            pltpu.SemaphoreType.DMA,
            pltpu.SemaphoreType.DMA,
            pltpu.SemaphoreType.DMA,
            pltpu.SemaphoreType.DMA,
        ],
    )(cur_x_pad, batch_idx_pad, embedding, tree_vecs_pad)


def _fused_body(x_ref, h_ref, o_ref, tv_ref, ptgt_ref, stgt_ref,
                wz1_ref, wz2_ref, wzb_ref,
                wr_ref, wrb_ref, ur_ref,
                wh1_ref, wh2_ref, whb_ref,
                w1_ref, w2_ref, wb_ref,
                u1_ref, u2_ref, u3_ref, ub_ref,
                wo_ref, wob_ref, us_ref, usb_ref,
                pl_out, sl_out, pa_out, sa_out):
    i = pl.program_id(0)

    ptgt = ptgt_ref[0]
    st = stgt_ref[0].astype(jnp.float32)

    x = x_ref[...]
    tv = tv_ref[...]
    h = h_ref[...]
    o = o_ref[...]

    sum_h = jnp.sum(h, axis=1)
    cur_o = jnp.sum(o, axis=1)
    z = jax.nn.sigmoid(
        jnp.dot(x, wz1_ref[...], preferred_element_type=jnp.float32)
        + jnp.dot(sum_h, wz2_ref[...], preferred_element_type=jnp.float32)
        + wzb_ref[...])
    r1 = jnp.dot(x, wr_ref[...], preferred_element_type=jnp.float32) + wrb_ref[...]
    hm = h.reshape(TT * NN, H)
    r2 = jnp.dot(hm, ur_ref[...], preferred_element_type=jnp.float32)
    r = jax.nn.sigmoid(r1[:, None, :] + r2.reshape(TT, NN, H))
    sum_gated = jnp.sum(r * h, axis=1)
    pre_h = jnp.tanh(
        jnp.dot(x, wh1_ref[...], preferred_element_type=jnp.float32)
        + jnp.dot(sum_gated, wh2_ref[...], preferred_element_type=jnp.float32)
        + whb_ref[...])
    new_h = (1.0 - z) * sum_h + z * pre_h

    pv = jax.nn.relu(
        jnp.dot(new_h, w1_ref[...], preferred_element_type=jnp.float32)
        + jnp.dot(tv, w2_ref[...], preferred_element_type=jnp.float32)
        + wb_ref[...])
    ps = jnp.dot(pv, wo_ref[...], preferred_element_type=jnp.float32) + wob_ref[...]
    m = jnp.max(ps, axis=1, keepdims=True)
    lse = m + jnp.log(jnp.sum(jnp.exp(ps - m), axis=1, keepdims=True))
    viota = jax.lax.broadcasted_iota(jnp.int32, (TT, V), 1)
    tmask = viota == ptgt
    tgt_logit = jnp.sum(jnp.where(tmask, ps, 0.0), axis=1, keepdims=True)
    pl_sum = jnp.sum(lse - tgt_logit, axis=0, keepdims=True)

    pa_sum = jnp.sum((tgt_logit == m).astype(jnp.float32), axis=0, keepdims=True)

    sv = jax.nn.relu(
        jnp.dot(x, u1_ref[...], preferred_element_type=jnp.float32)
        + jnp.dot(cur_o, u2_ref[...], preferred_element_type=jnp.float32)
        + jnp.dot(tv, u3_ref[...], preferred_element_type=jnp.float32)
        + ub_ref[...])
    ss = jnp.sum(sv * us_ref[...], axis=1, keepdims=True) + usb_ref[...]
    sp = jnp.maximum(ss, 0.0) + jnp.log1p(jnp.exp(-jnp.abs(ss)))
    sl_sum = jnp.sum(sp - ss * st, axis=0, keepdims=True)
    stops = (ss >= 0.0).astype(jnp.float32)
    sa_sum = jnp.sum((stops == st).astype(jnp.float32), axis=0, keepdims=True)

    @pl.when(i == 0)
    def _():
        pl_out[...] = jnp.zeros_like(pl_out)
        sl_out[...] = jnp.zeros_like(sl_out)
        pa_out[...] = jnp.zeros_like(pa_out)
        sa_out[...] = jnp.zeros_like(sa_out)

    pl_out[...] += pl_sum
    sl_out[...] += sl_sum
    pa_out[...] += pa_sum
    sa_out[...] += sa_sum


def kernel(cur_x, h_nei, o_nei, batch_idx, tree_vecs, pred_targets, stop_targets,
           embedding, Wz_w, Wz_b, Wr_w, Wr_b, Ur_w, Wh_w, Wh_b,
           W_w, W_b, U_w, U_b, Wo_w, Wo_b, Us_w, Us_b):
    cur_x_pad = jnp.pad(cur_x, (0, T_PAD - T))
    batch_idx_pad = jnp.pad(batch_idx, (0, T_PAD - T))
    tree_vecs_pad = jnp.pad(tree_vecs, ((0, 0), (0, LP - L)))
    x_all, tv_all = _sc_gather(cur_x_pad, batch_idx_pad, embedding,
                               tree_vecs_pad)

    idx3 = lambda a: a.reshape(NTILES, TT, 1)
    row = lambda b: b.reshape(1, -1)
    padw = lambda w: jnp.pad(w, ((0, LP - L), (0, 0)))

    tile_spec = lambda blk: pl.BlockSpec(blk, lambda i: (i, 0, 0))
    nei_spec = pl.BlockSpec((TT, NN, H), lambda i: (i, 0, 0))
    row_spec = lambda n: pl.BlockSpec((TT, n), lambda i: (i, 0))
    rep2 = lambda shape: pl.BlockSpec(shape, lambda i: (0, 0))

    args = (
        x_all, h_nei, o_nei, tv_all,
        idx3(pred_targets), idx3(stop_targets),
        Wz_w[:H], Wz_w[H:], row(Wz_b),
        Wr_w, row(Wr_b), Ur_w,
        Wh_w[:H], Wh_w[H:], row(Wh_b),
        W_w[:H], padw(W_w[H:]), row(W_b),
        U_w[:H], U_w[H:2 * H], padw(U_w[2 * H:]), row(U_b),
        Wo_w, row(Wo_b), Us_w.reshape(1, H), Us_b.reshape(1, 1),
    )
    in_specs = [
        row_spec(H), nei_spec, nei_spec, row_spec(LP),
        tile_spec((1, TT, 1)), tile_spec((1, TT, 1)),
        rep2((H, H)), rep2((H, H)), rep2((1, H)),
        rep2((H, H)), rep2((1, H)), rep2((H, H)),
        rep2((H, H)), rep2((H, H)), rep2((1, H)),
        rep2((H, H)), rep2((LP, H)), rep2((1, H)),
        rep2((H, H)), rep2((H, H)), rep2((LP, H)), rep2((1, H)),
        rep2((H, V)), rep2((1, V)), rep2((1, H)), rep2((1, 1)),
    ]
    out_specs = [pl.BlockSpec((1, 1), lambda i: (0, 0))] * 4
    out_shape = [jax.ShapeDtypeStruct((1, 1), jnp.float32)] * 4

    pls, sls, pas, sas = pl.pallas_call(
        _fused_body,
        grid=(NTILES,),
        in_specs=in_specs,
        out_specs=out_specs,
        out_shape=out_shape,
    )(*args)

    nB = jnp.float32(B)
    nT = jnp.float32(T)
    return (pls[0, 0] / nB, sls[0, 0] / nB, pas[0, 0] / nT, sas[0, 0] / nT)


# speedup vs baseline: 1.0440x; 1.0440x over previous
import functools

import jax
import jax.numpy as jnp
from jax import lax
from jax.experimental import pallas as pl
from jax.experimental.pallas import tpu as pltpu
from jax.experimental.pallas import tpu_sc as plsc

T = 50000
NN = 8
H = 128
L = 56
V = 780
B = 1024

TT = 2000
NTILES = T // TT

LP = 128
NC = 2
NS = 16
NW = NC * NS

NT1 = 12
NT2 = NTILES - NT1
BASE2 = NT1 * TT
CHUNK1 = 376
CHUNK2 = 408
NCHUNK = 2
ROWS1 = NW * CHUNK1 * NCHUNK
ROWS2 = NW * CHUNK2 * NCHUNK
IDXLEN = BASE2 + ROWS2


def _gather_body(base_off, chunk, curx_hbm, bidx_hbm, emb_hbm, tvs_hbm,
                 x_hbm, tv_hbm, idxx_v, idxb_v, xrows_v, tvrows_v,
                 semx, semb):
    wid = lax.axis_index("s") * NC + lax.axis_index("c")
    per_w = chunk * NCHUNK
    for c in range(NCHUNK):
        dst = wid * per_w + c * chunk
        src = base_off + dst
        pltpu.sync_copy(curx_hbm.at[pl.ds(src, chunk)], idxx_v)
        pltpu.sync_copy(bidx_hbm.at[pl.ds(src, chunk)], idxb_v)
        cpx = pltpu.async_copy(emb_hbm.at[idxx_v], xrows_v, semx)
        cpb = pltpu.async_copy(tvs_hbm.at[idxb_v], tvrows_v, semb)
        cpx.wait()
        cpb.wait()
        pltpu.sync_copy(xrows_v, x_hbm.at[pl.ds(dst, chunk)])
        pltpu.sync_copy(tvrows_v, tv_hbm.at[pl.ds(dst, chunk)])


def _sc_gather(cur_x_pad, batch_idx_pad, embedding, tree_vecs_pad,
               base_off, chunk):
    nrows = NW * chunk * NCHUNK
    mesh = plsc.VectorSubcoreMesh(core_axis_name="c", subcore_axis_name="s")
    return pl.kernel(
        functools.partial(_gather_body, base_off, chunk),
        out_type=[
            jax.ShapeDtypeStruct((nrows, H), jnp.float32),
            jax.ShapeDtypeStruct((nrows, LP), jnp.float32),
        ],
        mesh=mesh,
        scratch_types=[
            pltpu.VMEM((chunk,), jnp.int32),
            pltpu.VMEM((chunk,), jnp.int32),
            pltpu.VMEM((chunk, H), jnp.float32),
            pltpu.VMEM((chunk, LP), jnp.float32),
            pltpu.SemaphoreType.DMA,
            pltpu.SemaphoreType.DMA,
        ],
    )(cur_x_pad, batch_idx_pad, embedding, tree_vecs_pad)


def _fused_body(x_ref, h_ref, o_ref, tv_ref, ptgt_ref, stgt_ref,
                wz1_ref, wz2_ref, wzb_ref,
                wr_ref, wrb_ref, ur_ref,
                wh1_ref, wh2_ref, whb_ref,
                w1_ref, w2_ref, wb_ref,
                u1_ref, u2_ref, u3_ref, ub_ref,
                wo_ref, wob_ref, us_ref, usb_ref,
                pl_out, sl_out, pa_out, sa_out):
    i = pl.program_id(0)

    ptgt = ptgt_ref[0]
    st = stgt_ref[0].astype(jnp.float32)

    x = x_ref[...]
    tv = tv_ref[...]
    h = h_ref[...]
    o = o_ref[...]

    sum_h = jnp.sum(h, axis=1)
    cur_o = jnp.sum(o, axis=1)
    z = jax.nn.sigmoid(
        jnp.dot(x, wz1_ref[...], preferred_element_type=jnp.float32)
        + jnp.dot(sum_h, wz2_ref[...], preferred_element_type=jnp.float32)
        + wzb_ref[...])
    r1 = jnp.dot(x, wr_ref[...], preferred_element_type=jnp.float32) + wrb_ref[...]
    hm = h.reshape(TT * NN, H)
    r2 = jnp.dot(hm, ur_ref[...], preferred_element_type=jnp.float32)
    r = jax.nn.sigmoid(r1[:, None, :] + r2.reshape(TT, NN, H))
    sum_gated = jnp.sum(r * h, axis=1)
    pre_h = jnp.tanh(
        jnp.dot(x, wh1_ref[...], preferred_element_type=jnp.float32)
        + jnp.dot(sum_gated, wh2_ref[...], preferred_element_type=jnp.float32)
        + whb_ref[...])
    new_h = (1.0 - z) * sum_h + z * pre_h

    pv = jax.nn.relu(
        jnp.dot(new_h, w1_ref[...], preferred_element_type=jnp.float32)
        + jnp.dot(tv, w2_ref[...], preferred_element_type=jnp.float32)
        + wb_ref[...])
    ps = jnp.dot(pv, wo_ref[...], preferred_element_type=jnp.float32) + wob_ref[...]
    m = jnp.max(ps, axis=1, keepdims=True)
    lse = m + jnp.log(jnp.sum(jnp.exp(ps - m), axis=1, keepdims=True))
    viota = jax.lax.broadcasted_iota(jnp.int32, (TT, V), 1)
    tmask = viota == ptgt
    tgt_logit = jnp.sum(jnp.where(tmask, ps, 0.0), axis=1, keepdims=True)
    pl_sum = jnp.sum(lse - tgt_logit, axis=0, keepdims=True)

    pa_sum = jnp.sum((tgt_logit == m).astype(jnp.float32), axis=0, keepdims=True)

    sv = jax.nn.relu(
        jnp.dot(x, u1_ref[...], preferred_element_type=jnp.float32)
        + jnp.dot(cur_o, u2_ref[...], preferred_element_type=jnp.float32)
        + jnp.dot(tv, u3_ref[...], preferred_element_type=jnp.float32)
        + ub_ref[...])
    ss = jnp.sum(sv * us_ref[...], axis=1, keepdims=True) + usb_ref[...]
    sp = jnp.maximum(ss, 0.0) + jnp.log1p(jnp.exp(-jnp.abs(ss)))
    sl_sum = jnp.sum(sp - ss * st, axis=0, keepdims=True)
    stops = (ss >= 0.0).astype(jnp.float32)
    sa_sum = jnp.sum((stops == st).astype(jnp.float32), axis=0, keepdims=True)

    @pl.when(i == 0)
    def _():
        pl_out[...] = jnp.zeros_like(pl_out)
        sl_out[...] = jnp.zeros_like(sl_out)
        pa_out[...] = jnp.zeros_like(pa_out)
        sa_out[...] = jnp.zeros_like(sa_out)

    pl_out[...] += pl_sum
    sl_out[...] += sl_sum
    pa_out[...] += pa_sum
    sa_out[...] += sa_sum


def _fused_call(x_sl, tv_sl, h_nei, o_nei, ptgt3, stgt3, weights,
                tile_off, ntiles):
    off_spec = lambda blk: pl.BlockSpec(blk, lambda i: (i + tile_off, 0, 0))
    row_spec = lambda n: pl.BlockSpec((TT, n), lambda i: (i, 0))
    rep2 = lambda shape: pl.BlockSpec(shape, lambda i: (0, 0))

    args = (x_sl, h_nei, o_nei, tv_sl, ptgt3, stgt3, *weights)
    in_specs = [
        row_spec(H), off_spec((TT, NN, H)), off_spec((TT, NN, H)),
        row_spec(LP),
        off_spec((1, TT, 1)), off_spec((1, TT, 1)),
        rep2((H, H)), rep2((H, H)), rep2((1, H)),
        rep2((H, H)), rep2((1, H)), rep2((H, H)),
        rep2((H, H)), rep2((H, H)), rep2((1, H)),
        rep2((H, H)), rep2((LP, H)), rep2((1, H)),
        rep2((H, H)), rep2((H, H)), rep2((LP, H)), rep2((1, H)),
        rep2((H, V)), rep2((1, V)), rep2((1, H)), rep2((1, 1)),
    ]
    out_specs = [pl.BlockSpec((1, 1), lambda i: (0, 0))] * 4
    out_shape = [jax.ShapeDtypeStruct((1, 1), jnp.float32)] * 4

    return pl.pallas_call(
        _fused_body,
        grid=(ntiles,),
        in_specs=in_specs,
        out_specs=out_specs,
        out_shape=out_shape,
    )(*args)


def kernel(cur_x, h_nei, o_nei, batch_idx, tree_vecs, pred_targets, stop_targets,
           embedding, Wz_w, Wz_b, Wr_w, Wr_b, Ur_w, Wh_w, Wh_b,
           W_w, W_b, U_w, U_b, Wo_w, Wo_b, Us_w, Us_b):
    cur_x_pad = jnp.pad(cur_x, (0, IDXLEN - T))
    batch_idx_pad = jnp.pad(batch_idx, (0, IDXLEN - T))
    tree_vecs_pad = jnp.pad(tree_vecs, ((0, 0), (0, LP - L)))
    x1, tv1 = _sc_gather(cur_x_pad, batch_idx_pad, embedding,
                         tree_vecs_pad, 0, CHUNK1)
    x2, tv2 = _sc_gather(cur_x_pad, batch_idx_pad, embedding,
                         tree_vecs_pad, BASE2, CHUNK2)

    idx3 = lambda a: a.reshape(NTILES, TT, 1)
    row = lambda b: b.reshape(1, -1)
    padw = lambda w: jnp.pad(w, ((0, LP - L), (0, 0)))

    weights = (
        Wz_w[:H], Wz_w[H:], row(Wz_b),
        Wr_w, row(Wr_b), Ur_w,
        Wh_w[:H], Wh_w[H:], row(Wh_b),
        W_w[:H], padw(W_w[H:]), row(W_b),
        U_w[:H], U_w[H:2 * H], padw(U_w[2 * H:]), row(U_b),
        Wo_w, row(Wo_b), Us_w.reshape(1, H), Us_b.reshape(1, 1),
    )
    ptgt3 = idx3(pred_targets)
    stgt3 = idx3(stop_targets)

    outs1 = _fused_call(x1, tv1, h_nei, o_nei, ptgt3, stgt3, weights,
                        0, NT1)
    outs2 = _fused_call(x2, tv2, h_nei, o_nei, ptgt3, stgt3, weights,
                        NT1, NT2)
    pls, sls, pas, sas = [a + b for a, b in zip(outs1, outs2)]

    nB = jnp.float32(B)
    nT = jnp.float32(T)
    return (pls[0, 0] / nB, sls[0, 0] / nB, pas[0, 0] / nT, sas[0, 0] / nT)


# speedup vs baseline: 1.0896x; 1.0436x over previous
import functools

import jax
import jax.numpy as jnp
from jax import lax
from jax.experimental import pallas as pl
from jax.experimental.pallas import tpu as pltpu
from jax.experimental.pallas import tpu_sc as plsc

T = 50000
NN = 8
H = 128
L = 56
V = 780
B = 1024

TT = 2000
NTILES = T // TT

LP = 128
NC = 2
NS = 16
NW = NC * NS
CHUNK = 392
NCHUNK = 4
PER_W = CHUNK * NCHUNK
T_PAD = NW * PER_W


def _gather_body(curx_hbm, bidx_hbm, emb_hbm, tvs_hbm, x_hbm, tv_hbm,
                 idxx_v, idxb_v, xrows_v, tvrows_v, semx, semb):
    wid = lax.axis_index("s") * NC + lax.axis_index("c")
    base0 = wid * PER_W
    for c in range(NCHUNK):
        base = base0 + c * CHUNK
        pltpu.sync_copy(curx_hbm.at[pl.ds(base, CHUNK)], idxx_v)
        pltpu.sync_copy(bidx_hbm.at[pl.ds(base, CHUNK)], idxb_v)
        cpx = pltpu.async_copy(emb_hbm.at[idxx_v], xrows_v, semx)
        cpb = pltpu.async_copy(tvs_hbm.at[idxb_v], tvrows_v, semb)
        cpx.wait()
        cpb.wait()
        pltpu.sync_copy(xrows_v, x_hbm.at[pl.ds(base, CHUNK)])
        pltpu.sync_copy(tvrows_v, tv_hbm.at[pl.ds(base, CHUNK)])


def _sc_gather(cur_x_pad, batch_idx_pad, embedding, tree_vecs_pad):
    mesh = plsc.VectorSubcoreMesh(core_axis_name="c", subcore_axis_name="s")
    return pl.kernel(
        _gather_body,
        out_type=[
            jax.ShapeDtypeStruct((T_PAD, H), jnp.float32),
            jax.ShapeDtypeStruct((T_PAD, LP), jnp.float32),
        ],
        mesh=mesh,
        scratch_types=[
            pltpu.VMEM((CHUNK,), jnp.int32),
            pltpu.VMEM((CHUNK,), jnp.int32),
            pltpu.VMEM((CHUNK, H), jnp.float32),
            pltpu.VMEM((CHUNK, LP), jnp.float32),
            pltpu.SemaphoreType.DMA,
            pltpu.SemaphoreType.DMA,
        ],
    )(cur_x_pad, batch_idx_pad, embedding, tree_vecs_pad)


def _fused_body(x_ref, h_ref, o_ref, tv_ref, ptgt_ref, stgt_ref,
                wz1_ref, wz2_ref, wzb_ref,
                wr_ref, wrb_ref, ur_ref,
                wh1_ref, wh2_ref, whb_ref,
                w1_ref, w2_ref, wb_ref,
                u1_ref, u2_ref, u3_ref, ub_ref,
                wo_ref, wob_ref, us_ref, usb_ref,
                pl_out, sl_out, pa_out, sa_out):
    i = pl.program_id(0)

    ptgt = ptgt_ref[0]
    st = stgt_ref[0].astype(jnp.float32)

    x = x_ref[...]
    tv = tv_ref[...]
    h = h_ref[...]
    o = o_ref[...]

    sum_h = jnp.sum(h, axis=1)
    cur_o = jnp.sum(o, axis=1)
    z = jax.nn.sigmoid(
        jnp.dot(x, wz1_ref[...], preferred_element_type=jnp.float32)
        + jnp.dot(sum_h, wz2_ref[...], preferred_element_type=jnp.float32)
        + wzb_ref[...])
    r1 = jnp.dot(x, wr_ref[...], preferred_element_type=jnp.float32) + wrb_ref[...]
    hm = h.reshape(TT * NN, H)
    r2 = jnp.dot(hm, ur_ref[...], preferred_element_type=jnp.float32)
    r = jax.nn.sigmoid(r1[:, None, :] + r2.reshape(TT, NN, H))
    sum_gated = jnp.sum(r * h, axis=1)
    pre_h = jnp.tanh(
        jnp.dot(x, wh1_ref[...], preferred_element_type=jnp.float32)
        + jnp.dot(sum_gated, wh2_ref[...], preferred_element_type=jnp.float32)
        + whb_ref[...])
    new_h = (1.0 - z) * sum_h + z * pre_h

    pv = jax.nn.relu(
        jnp.dot(new_h, w1_ref[...], preferred_element_type=jnp.float32)
        + jnp.dot(tv, w2_ref[...], preferred_element_type=jnp.float32)
        + wb_ref[...])
    ps = jnp.dot(pv, wo_ref[...], preferred_element_type=jnp.float32) + wob_ref[...]
    m = jnp.max(ps, axis=1, keepdims=True)
    lse = m + jnp.log(jnp.sum(jnp.exp(ps - m), axis=1, keepdims=True))
    viota = jax.lax.broadcasted_iota(jnp.int32, (TT, V), 1)
    tmask = viota == ptgt
    tgt_logit = jnp.sum(jnp.where(tmask, ps, 0.0), axis=1, keepdims=True)
    pl_sum = jnp.sum(lse - tgt_logit, axis=0, keepdims=True)

    pa_sum = jnp.sum((tgt_logit == m).astype(jnp.float32), axis=0, keepdims=True)

    sv = jax.nn.relu(
        jnp.dot(x, u1_ref[...], preferred_element_type=jnp.float32)
        + jnp.dot(cur_o, u2_ref[...], preferred_element_type=jnp.float32)
        + jnp.dot(tv, u3_ref[...], preferred_element_type=jnp.float32)
        + ub_ref[...])
    ss = jnp.sum(sv * us_ref[...], axis=1, keepdims=True) + usb_ref[...]
    sp = jnp.maximum(ss, 0.0) + jnp.log1p(jnp.exp(-jnp.abs(ss)))
    sl_sum = jnp.sum(sp - ss * st, axis=0, keepdims=True)
    stops = (ss >= 0.0).astype(jnp.float32)
    sa_sum = jnp.sum((stops == st).astype(jnp.float32), axis=0, keepdims=True)

    @pl.when(i == 0)
    def _():
        pl_out[...] = jnp.zeros_like(pl_out)
        sl_out[...] = jnp.zeros_like(sl_out)
        pa_out[...] = jnp.zeros_like(pa_out)
        sa_out[...] = jnp.zeros_like(sa_out)

    pl_out[...] += pl_sum
    sl_out[...] += sl_sum
    pa_out[...] += pa_sum
    sa_out[...] += sa_sum


def kernel(cur_x, h_nei, o_nei, batch_idx, tree_vecs, pred_targets, stop_targets,
           embedding, Wz_w, Wz_b, Wr_w, Wr_b, Ur_w, Wh_w, Wh_b,
           W_w, W_b, U_w, U_b, Wo_w, Wo_b, Us_w, Us_b):
    cur_x_pad = jnp.pad(cur_x, (0, T_PAD - T))
    batch_idx_pad = jnp.pad(batch_idx, (0, T_PAD - T))
    tree_vecs_pad = jnp.pad(tree_vecs, ((0, 0), (0, LP - L)))
    x_all, tv_all = _sc_gather(cur_x_pad, batch_idx_pad, embedding,
                               tree_vecs_pad)

    idx3 = lambda a: a.reshape(NTILES, TT, 1)
    row = lambda b: b.reshape(1, -1)
    padw = lambda w: jnp.pad(w, ((0, LP - L), (0, 0)))

    tile_spec = lambda blk: pl.BlockSpec(blk, lambda i: (i, 0, 0))
    nei_spec = pl.BlockSpec((TT, NN, H), lambda i: (i, 0, 0))
    row_spec = lambda n: pl.BlockSpec((TT, n), lambda i: (i, 0))
    rep2 = lambda shape: pl.BlockSpec(shape, lambda i: (0, 0))

    args = (
        x_all, h_nei, o_nei, tv_all,
        idx3(pred_targets), idx3(stop_targets),
        Wz_w[:H], Wz_w[H:], row(Wz_b),
        Wr_w, row(Wr_b), Ur_w,
        Wh_w[:H], Wh_w[H:], row(Wh_b),
        W_w[:H], padw(W_w[H:]), row(W_b),
        U_w[:H], U_w[H:2 * H], padw(U_w[2 * H:]), row(U_b),
        Wo_w, row(Wo_b), Us_w.reshape(1, H), Us_b.reshape(1, 1),
    )
    in_specs = [
        row_spec(H), nei_spec, nei_spec, row_spec(LP),
        tile_spec((1, TT, 1)), tile_spec((1, TT, 1)),
        rep2((H, H)), rep2((H, H)), rep2((1, H)),
        rep2((H, H)), rep2((1, H)), rep2((H, H)),
        rep2((H, H)), rep2((H, H)), rep2((1, H)),
        rep2((H, H)), rep2((LP, H)), rep2((1, H)),
        rep2((H, H)), rep2((H, H)), rep2((LP, H)), rep2((1, H)),
        rep2((H, V)), rep2((1, V)), rep2((1, H)), rep2((1, 1)),
    ]
    out_specs = [pl.BlockSpec((1, 1), lambda i: (0, 0))] * 4
    out_shape = [jax.ShapeDtypeStruct((1, 1), jnp.float32)] * 4

    pls, sls, pas, sas = pl.pallas_call(
        _fused_body,
        grid=(NTILES,),
        in_specs=in_specs,
        out_specs=out_specs,
        out_shape=out_shape,
    )(*args)

    nB = jnp.float32(B)
    nT = jnp.float32(T)
    return (pls[0, 0] / nB, sls[0, 0] / nB, pas[0, 0] / nT, sas[0, 0] / nT)
